# baseline (speedup 1.0000x reference)
import functools

import jax
import jax.numpy as jnp
from jax import lax
from jax.experimental import pallas as pl
from jax.experimental.pallas import tpu as pltpu
from jax.experimental.pallas import tpu_sc as plsc

_NC = 2
_NS = 16
_NW = _NC * _NS
_C = 50
_K = 4
_NZ = 5


def _sc_mesh():
    return plsc.VectorSubcoreMesh(
        core_axis_name="c", subcore_axis_name="s",
        num_cores=_NC, num_subcores=_NS)


_SC_PARAMS = pltpu.CompilerParams(use_tc_tiling_on_sc=False)


def _make_deg(n_nodes, n_chunks):
    cpt = n_chunks // _NW
    zblk = n_nodes // _NZ

    @functools.partial(
        pl.kernel,
        out_type=jax.ShapeDtypeStruct((_NC, n_nodes), jnp.float32),
        mesh=_sc_mesh(),
        scratch_types=[
            pltpu.VMEM((cpt, _C), jnp.int32),
            pltpu.VMEM((128,), jnp.float32),
            pltpu.VMEM_SHARED((n_nodes,), jnp.float32),
            pltpu.SemaphoreType.DMA,
        ],
        compiler_params=_SC_PARAMS,
    )
    def deg_kernel(dst_hbm, zeros_hbm, out_hbm, dst_v, ones_v, deg_sh, sem):
        c = lax.axis_index("c")
        s = lax.axis_index("s")
        w = s * _NC + c
        for i in range(8):
            ones_v[pl.ds(i * 16, 16)] = jnp.ones((16,), jnp.float32)
        pltpu.sync_copy(dst_hbm.at[pl.ds(w * cpt, cpt)], dst_v)

        @pl.when(s < _NZ)
        def _zero():
            pltpu.sync_copy(zeros_hbm.at[pl.ds(s * zblk, zblk)],
                            deg_sh.at[pl.ds(s * zblk, zblk)])

        plsc.subcore_barrier()

        def body(j, carry):
            pltpu.sync_copy(ones_v.at[pl.ds(0, _C)], deg_sh.at[dst_v.at[j]],
                            add=True)
            return carry

        lax.fori_loop(0, cpt, body, 0)
        plsc.subcore_barrier()

        @pl.when(s < _NZ)
        def _emit():
            pltpu.sync_copy(deg_sh.at[pl.ds(s * zblk, zblk)],
                            out_hbm.at[c, pl.ds(s * zblk, zblk)])

    return deg_kernel


def _make_agg(n_nodes, n_chunks, d):
    cpt = n_chunks // _NW
    rpt = n_nodes // _NS

    @functools.partial(
        pl.kernel,
        out_type=jax.ShapeDtypeStruct((_NC, n_nodes, d), jnp.float32),
        mesh=_sc_mesh(),
        scratch_types=[
            pltpu.VMEM_SHARED((n_nodes, d), jnp.float32),
            pltpu.VMEM((cpt, _C), jnp.int32),
            pltpu.VMEM((cpt, _C), jnp.int32),
            [pltpu.VMEM((_C, d), jnp.float32) for _ in range(_K)],
            pltpu.SemaphoreType.DMA((_K,)),
            pltpu.SemaphoreType.DMA((_K,)),
        ],
        compiler_params=_SC_PARAMS,
    )
    def agg_kernel(x_hbm, src_hbm, dst_hbm, zeros_hbm, out_hbm,
                   acc_sh, src_v, dst_v, rows, gsem, ssem):
        c = lax.axis_index("c")
        s = lax.axis_index("s")
        w = s * _NC + c
        pltpu.sync_copy(src_hbm.at[pl.ds(w * cpt, cpt)], src_v)
        pltpu.sync_copy(dst_hbm.at[pl.ds(w * cpt, cpt)], dst_v)
        pltpu.sync_copy(zeros_hbm.at[pl.ds(s * rpt, rpt)],
                        acc_sh.at[pl.ds(s * rpt, rpt)])
        plsc.subcore_barrier()

        for b in range(_K):
            pltpu.async_copy(x_hbm.at[src_v.at[b]], rows[b], gsem.at[b])

        def round_body(r, carry):
            for b in range(_K):
                j = r * _K + b
                b2 = (b + 2) % _K
                pltpu.make_async_copy(x_hbm.at[src_v.at[j]], rows[b],
                                      gsem.at[b]).wait()
                pltpu.async_copy(rows[b], acc_sh.at[dst_v.at[j]], ssem.at[b],
                                 add=True)

                @pl.when(jnp.logical_and(j >= 2, j + _K - 2 < cpt))
                def _refill():
                    pltpu.make_async_copy(rows[b2], acc_sh.at[dst_v.at[j - 2]],
                                          ssem.at[b2]).wait()
                    pltpu.async_copy(x_hbm.at[src_v.at[j + _K - 2]], rows[b2],
                                    gsem.at[b2])
            return carry

        lax.fori_loop(0, cpt // _K, round_body, 0)
        for b in range(_K):
            pltpu.make_async_copy(rows[b], acc_sh.at[dst_v.at[cpt - _K + b]],
                                  ssem.at[b]).wait()
        plsc.subcore_barrier()
        pltpu.sync_copy(acc_sh.at[pl.ds(s * rpt, rpt)],
                        out_hbm.at[c, pl.ds(s * rpt, rpt)])

    return agg_kernel


def _norm_from_deg(degr):
    dsum = degr[0] + degr[1]
    return jnp.where(dsum > 0, lax.rsqrt(jnp.maximum(dsum, 1.0)), 0.0)


def _scale_kernel(deg_ref, x_ref, o_ref):
    o_ref[...] = x_ref[...] * _norm_from_deg(deg_ref[...])


def _layer1_kernel(deg_ref, p_ref, w_ref, b_ref, o_ref):
    norm = _norm_from_deg(deg_ref[...])
    agg = (p_ref[0] + p_ref[1]) * norm
    h = jnp.dot(agg, w_ref[...], preferred_element_type=jnp.float32) + b_ref[...]
    o_ref[...] = jnp.maximum(h, 0.0) * norm


def _layer2_kernel(deg_ref, p_ref, w2_ref, b2_ref, w3_ref, b3_ref,
                   o_ref, acc_ref, *, nblocks, n_nodes):
    i = pl.program_id(0)
    norm = _norm_from_deg(deg_ref[...])
    agg = (p_ref[0] + p_ref[1]) * norm
    h = jnp.dot(agg, w2_ref[...], preferred_element_type=jnp.float32) + b2_ref[...]
    h = jnp.maximum(h, 0.0)
    csum = jnp.sum(h, axis=0, keepdims=True)

    @pl.when(i == 0)
    def _init():
        acc_ref[...] = jnp.zeros_like(acc_ref)

    acc_ref[...] += csum

    @pl.when(i == nblocks - 1)
    def _final():
        hg = acc_ref[...] * (1.0 / n_nodes)
        o_ref[...] = (jnp.dot(hg, w3_ref[...], preferred_element_type=jnp.float32)
                      + b3_ref[...])


def kernel(x, edge_index, W1, b1, W2, b2, W3, b3):
    n, din = x.shape
    e = edge_index.shape[1]
    hid = W1.shape[1]
    out2 = W2.shape[1]
    ncls = W3.shape[1]
    nch = e // _C

    src2d = edge_index[0].reshape(nch, _C)
    dst2d = edge_index[1].reshape(nch, _C)
    zeros1 = jnp.zeros((n,), jnp.float32)
    zerosf = jnp.zeros((n, din), jnp.float32)

    deg = _make_deg(n, nch)(dst2d, zeros1)
    degcol = deg.reshape(_NC, n, 1)

    BLK = 1000
    gridn = n // BLK

    xs = pl.pallas_call(
        _scale_kernel,
        grid=(gridn,),
        in_specs=[
            pl.BlockSpec((_NC, BLK, 1), lambda i: (0, i, 0)),
            pl.BlockSpec((BLK, din), lambda i: (i, 0)),
        ],
        out_specs=pl.BlockSpec((BLK, din), lambda i: (i, 0)),
        out_shape=jax.ShapeDtypeStruct((n, din), jnp.float32),
    )(degcol, x)

    p1 = _make_agg(n, nch, din)(xs, src2d, dst2d, zerosf)

    h1s = pl.pallas_call(
        _layer1_kernel,
        grid=(gridn,),
        in_specs=[
            pl.BlockSpec((_NC, BLK, 1), lambda i: (0, i, 0)),
            pl.BlockSpec((_NC, BLK, din), lambda i: (0, i, 0)),
            pl.BlockSpec((din, hid), lambda i: (0, 0)),
            pl.BlockSpec((1, hid), lambda i: (0, 0)),
        ],
        out_specs=pl.BlockSpec((BLK, hid), lambda i: (i, 0)),
        out_shape=jax.ShapeDtypeStruct((n, hid), jnp.float32),
    )(degcol, p1, W1, b1.reshape(1, hid))

    p2 = _make_agg(n, nch, hid)(h1s, src2d, dst2d, zerosf)

    y = pl.pallas_call(
        functools.partial(_layer2_kernel, nblocks=gridn, n_nodes=n),
        grid=(gridn,),
        in_specs=[
            pl.BlockSpec((_NC, BLK, 1), lambda i: (0, i, 0)),
            pl.BlockSpec((_NC, BLK, hid), lambda i: (0, i, 0)),
            pl.BlockSpec((hid, out2), lambda i: (0, 0)),
            pl.BlockSpec((1, out2), lambda i: (0, 0)),
            pl.BlockSpec((out2, ncls), lambda i: (0, 0)),
            pl.BlockSpec((1, ncls), lambda i: (0, 0)),
        ],
        out_specs=pl.BlockSpec((1, ncls), lambda i: (0, 0)),
        out_shape=jax.ShapeDtypeStruct((1, ncls), jnp.float32),
        scratch_shapes=[pltpu.VMEM((1, out2), jnp.float32)],
    )(degcol, p2, W2, b2.reshape(1, out2), W3, b3.reshape(1, ncls))

    return y


# speedup vs baseline: 1.2471x; 1.2471x over previous
import functools

import jax
import jax.numpy as jnp
from jax import lax
from jax.experimental import pallas as pl
from jax.experimental.pallas import tpu as pltpu
from jax.experimental.pallas import tpu_sc as plsc

_NC = 2
_NS = 16
_NW = _NC * _NS
_C = 100
_K = 4
_NZ = 5


def _sc_mesh():
    return plsc.VectorSubcoreMesh(
        core_axis_name="c", subcore_axis_name="s",
        num_cores=_NC, num_subcores=_NS)


_SC_PARAMS = pltpu.CompilerParams(use_tc_tiling_on_sc=False)


def _make_deg(n_nodes, n_chunks):
    cpt = n_chunks // _NW
    zblk = n_nodes // _NZ

    @functools.partial(
        pl.kernel,
        out_type=jax.ShapeDtypeStruct((_NC, n_nodes), jnp.float32),
        mesh=_sc_mesh(),
        scratch_types=[
            pltpu.VMEM((cpt, _C), jnp.int32),
            pltpu.VMEM((128,), jnp.float32),
            pltpu.VMEM_SHARED((n_nodes,), jnp.float32),
            pltpu.SemaphoreType.DMA,
        ],
        compiler_params=_SC_PARAMS,
    )
    def deg_kernel(dst_hbm, zeros_hbm, out_hbm, dst_v, ones_v, deg_sh, sem):
        c = lax.axis_index("c")
        s = lax.axis_index("s")
        w = s * _NC + c
        for i in range(8):
            ones_v[pl.ds(i * 16, 16)] = jnp.ones((16,), jnp.float32)
        pltpu.sync_copy(dst_hbm.at[pl.ds(w * cpt, cpt)], dst_v)

        @pl.when(s < _NZ)
        def _zero():
            pltpu.sync_copy(zeros_hbm.at[pl.ds(s * zblk, zblk)],
                            deg_sh.at[pl.ds(s * zblk, zblk)])

        plsc.subcore_barrier()

        def body(j, carry):
            pltpu.sync_copy(ones_v.at[pl.ds(0, _C)], deg_sh.at[dst_v.at[j]],
                            add=True)
            return carry

        lax.fori_loop(0, cpt, body, 0)
        plsc.subcore_barrier()

        @pl.when(s < _NZ)
        def _emit():
            pltpu.sync_copy(deg_sh.at[pl.ds(s * zblk, zblk)],
                            out_hbm.at[c, pl.ds(s * zblk, zblk)])

    return deg_kernel


def _make_agg(n_nodes, n_chunks, d):
    cpt = n_chunks // _NW
    rpt = n_nodes // _NS

    @functools.partial(
        pl.kernel,
        out_type=jax.ShapeDtypeStruct((_NC, n_nodes, d), jnp.bfloat16),
        mesh=_sc_mesh(),
        scratch_types=[
            pltpu.VMEM_SHARED((n_nodes, d), jnp.bfloat16),
            pltpu.VMEM((cpt, _C), jnp.int32),
            pltpu.VMEM((cpt, _C), jnp.int32),
            [pltpu.VMEM((_C, d), jnp.bfloat16) for _ in range(_K)],
            pltpu.SemaphoreType.DMA((_K,)),
            pltpu.SemaphoreType.DMA((_K,)),
        ],
        compiler_params=_SC_PARAMS,
    )
    def agg_kernel(x_hbm, src_hbm, dst_hbm, zeros_hbm, out_hbm,
                   acc_sh, src_v, dst_v, rows, gsem, ssem):
        c = lax.axis_index("c")
        s = lax.axis_index("s")
        w = s * _NC + c
        pltpu.sync_copy(src_hbm.at[pl.ds(w * cpt, cpt)], src_v)
        pltpu.sync_copy(dst_hbm.at[pl.ds(w * cpt, cpt)], dst_v)
        pltpu.sync_copy(zeros_hbm.at[pl.ds(s * rpt, rpt)],
                        acc_sh.at[pl.ds(s * rpt, rpt)])
        plsc.subcore_barrier()

        for b in range(_K):
            pltpu.async_copy(x_hbm.at[src_v.at[b]], rows[b], gsem.at[b])

        def round_body(r, carry):
            for b in range(_K):
                j = r * _K + b
                b2 = (b + 2) % _K
                pltpu.make_async_copy(x_hbm.at[src_v.at[j]], rows[b],
                                      gsem.at[b]).wait()
                pltpu.async_copy(rows[b], acc_sh.at[dst_v.at[j]], ssem.at[b],
                                 add=True)

                @pl.when(jnp.logical_and(j >= 2, j + _K - 2 < cpt))
                def _refill():
                    pltpu.make_async_copy(rows[b2], acc_sh.at[dst_v.at[j - 2]],
                                          ssem.at[b2]).wait()
                    pltpu.async_copy(x_hbm.at[src_v.at[j + _K - 2]], rows[b2],
                                    gsem.at[b2])
            return carry

        lax.fori_loop(0, cpt // _K, round_body, 0)
        for b in range(_K):
            pltpu.make_async_copy(rows[b], acc_sh.at[dst_v.at[cpt - _K + b]],
                                  ssem.at[b]).wait()
        plsc.subcore_barrier()
        pltpu.sync_copy(acc_sh.at[pl.ds(s * rpt, rpt)],
                        out_hbm.at[c, pl.ds(s * rpt, rpt)])

    return agg_kernel


def _norm_from_deg(degr):
    dsum = degr[0] + degr[1]
    return jnp.where(dsum > 0, lax.rsqrt(jnp.maximum(dsum, 1.0)), 0.0)


def _scale_kernel(deg_ref, x_ref, o_ref):
    o_ref[...] = (x_ref[...] * _norm_from_deg(deg_ref[...])).astype(jnp.bfloat16)


def _layer1_kernel(deg_ref, p_ref, w_ref, b_ref, o_ref):
    norm = _norm_from_deg(deg_ref[...])
    agg = (p_ref[0].astype(jnp.float32) + p_ref[1].astype(jnp.float32)) * norm
    h = jnp.dot(agg, w_ref[...], preferred_element_type=jnp.float32) + b_ref[...]
    o_ref[...] = (jnp.maximum(h, 0.0) * norm).astype(jnp.bfloat16)


def _layer2_kernel(deg_ref, p_ref, w2_ref, b2_ref, w3_ref, b3_ref,
                   o_ref, acc_ref, *, nblocks, n_nodes):
    i = pl.program_id(0)
    norm = _norm_from_deg(deg_ref[...])
    agg = (p_ref[0].astype(jnp.float32) + p_ref[1].astype(jnp.float32)) * norm
    h = jnp.dot(agg, w2_ref[...], preferred_element_type=jnp.float32) + b2_ref[...]
    h = jnp.maximum(h, 0.0)
    csum = jnp.sum(h, axis=0, keepdims=True)

    @pl.when(i == 0)
    def _init():
        acc_ref[...] = jnp.zeros_like(acc_ref)

    acc_ref[...] += csum

    @pl.when(i == nblocks - 1)
    def _final():
        hg = acc_ref[...] * (1.0 / n_nodes)
        o_ref[...] = (jnp.dot(hg, w3_ref[...], preferred_element_type=jnp.float32)
                      + b3_ref[...])


def kernel(x, edge_index, W1, b1, W2, b2, W3, b3):
    n, din = x.shape
    e = edge_index.shape[1]
    hid = W1.shape[1]
    out2 = W2.shape[1]
    ncls = W3.shape[1]
    nch = e // _C

    src2d = edge_index[0].reshape(nch, _C)
    dst2d = edge_index[1].reshape(nch, _C)
    zeros1 = jnp.zeros((n,), jnp.float32)
    zerosf = jnp.zeros((n, din), jnp.bfloat16)

    deg = _make_deg(n, nch)(dst2d, zeros1)
    degcol = deg.reshape(_NC, n, 1)

    BLK = 1000
    gridn = n // BLK

    xs = pl.pallas_call(
        _scale_kernel,
        grid=(gridn,),
        in_specs=[
            pl.BlockSpec((_NC, BLK, 1), lambda i: (0, i, 0)),
            pl.BlockSpec((BLK, din), lambda i: (i, 0)),
        ],
        out_specs=pl.BlockSpec((BLK, din), lambda i: (i, 0)),
        out_shape=jax.ShapeDtypeStruct((n, din), jnp.bfloat16),
    )(degcol, x)

    p1 = _make_agg(n, nch, din)(xs, src2d, dst2d, zerosf)

    h1s = pl.pallas_call(
        _layer1_kernel,
        grid=(gridn,),
        in_specs=[
            pl.BlockSpec((_NC, BLK, 1), lambda i: (0, i, 0)),
            pl.BlockSpec((_NC, BLK, din), lambda i: (0, i, 0)),
            pl.BlockSpec((din, hid), lambda i: (0, 0)),
            pl.BlockSpec((1, hid), lambda i: (0, 0)),
        ],
        out_specs=pl.BlockSpec((BLK, hid), lambda i: (i, 0)),
        out_shape=jax.ShapeDtypeStruct((n, hid), jnp.bfloat16),
    )(degcol, p1, W1, b1.reshape(1, hid))

    p2 = _make_agg(n, nch, hid)(h1s, src2d, dst2d, zerosf)

    y = pl.pallas_call(
        functools.partial(_layer2_kernel, nblocks=gridn, n_nodes=n),
        grid=(gridn,),
        in_specs=[
            pl.BlockSpec((_NC, BLK, 1), lambda i: (0, i, 0)),
            pl.BlockSpec((_NC, BLK, hid), lambda i: (0, i, 0)),
            pl.BlockSpec((hid, out2), lambda i: (0, 0)),
            pl.BlockSpec((1, out2), lambda i: (0, 0)),
            pl.BlockSpec((out2, ncls), lambda i: (0, 0)),
            pl.BlockSpec((1, ncls), lambda i: (0, 0)),
        ],
        out_specs=pl.BlockSpec((1, ncls), lambda i: (0, 0)),
        out_shape=jax.ShapeDtypeStruct((1, ncls), jnp.float32),
        scratch_shapes=[pltpu.VMEM((1, out2), jnp.float32)],
    )(degcol, p2, W2, b2.reshape(1, out2), W3, b3.reshape(1, ncls))

    return y


# speedup vs baseline: 1.3278x; 1.0647x over previous
import functools

import jax
import jax.numpy as jnp
from jax import lax
from jax.experimental import pallas as pl
from jax.experimental.pallas import tpu as pltpu
from jax.experimental.pallas import tpu_sc as plsc

_NC = 2
_NS = 16
_NW = _NC * _NS
_C = 100
_K = 4
_NZ = 5


def _sc_mesh():
    return plsc.VectorSubcoreMesh(
        core_axis_name="c", subcore_axis_name="s",
        num_cores=_NC, num_subcores=_NS)


_SC_PARAMS = pltpu.CompilerParams(use_tc_tiling_on_sc=False)


def _make_deg(n_nodes, n_chunks):
    cpt = n_chunks // _NW
    zblk = n_nodes // _NZ

    @functools.partial(
        pl.kernel,
        out_type=jax.ShapeDtypeStruct((_NC, n_nodes), jnp.float32),
        mesh=_sc_mesh(),
        scratch_types=[
            pltpu.VMEM((cpt, _C), jnp.int32),
            pltpu.VMEM((128,), jnp.float32),
            pltpu.VMEM_SHARED((n_nodes,), jnp.float32),
            pltpu.SemaphoreType.DMA,
        ],
        compiler_params=_SC_PARAMS,
    )
    def deg_kernel(dst_hbm, zeros_hbm, out_hbm, dst_v, ones_v, deg_sh, sem):
        c = lax.axis_index("c")
        s = lax.axis_index("s")
        w = s * _NC + c
        for i in range(8):
            ones_v[pl.ds(i * 16, 16)] = jnp.ones((16,), jnp.float32)
        pltpu.sync_copy(dst_hbm.at[pl.ds(w * cpt, cpt)], dst_v)

        @pl.when(s < _NZ)
        def _zero():
            pltpu.sync_copy(zeros_hbm, deg_sh.at[pl.ds(s * zblk, zblk)])

        plsc.subcore_barrier()

        def body(j, carry):
            pltpu.sync_copy(ones_v.at[pl.ds(0, _C)], deg_sh.at[dst_v.at[j]],
                            add=True)
            return carry

        lax.fori_loop(0, cpt, body, 0)
        plsc.subcore_barrier()

        @pl.when(s < _NZ)
        def _emit():
            pltpu.sync_copy(deg_sh.at[pl.ds(s * zblk, zblk)],
                            out_hbm.at[c, pl.ds(s * zblk, zblk)])

    return deg_kernel


def _make_agg(n_nodes, n_chunks, d):
    cpt = n_chunks // _NW
    rpt = n_nodes // _NS

    @functools.partial(
        pl.kernel,
        out_type=jax.ShapeDtypeStruct((_NC, n_nodes, d), jnp.bfloat16),
        mesh=_sc_mesh(),
        scratch_types=[
            pltpu.VMEM_SHARED((n_nodes, d), jnp.bfloat16),
            pltpu.VMEM((cpt, _C), jnp.int32),
            pltpu.VMEM((cpt, _C), jnp.int32),
            [pltpu.VMEM((_C, d), jnp.bfloat16) for _ in range(_K)],
            pltpu.SemaphoreType.DMA((_K,)),
            pltpu.SemaphoreType.DMA((_K,)),
        ],
        compiler_params=_SC_PARAMS,
    )
    def agg_kernel(x_hbm, src_hbm, dst_hbm, zeros_hbm, out_hbm,
                   acc_sh, src_v, dst_v, rows, gsem, ssem):
        c = lax.axis_index("c")
        s = lax.axis_index("s")
        w = s * _NC + c
        pltpu.sync_copy(src_hbm.at[pl.ds(w * cpt, cpt)], src_v)
        pltpu.sync_copy(dst_hbm.at[pl.ds(w * cpt, cpt)], dst_v)
        pltpu.sync_copy(zeros_hbm, acc_sh.at[pl.ds(s * rpt, rpt)])
        plsc.subcore_barrier()

        for b in range(_K):
            pltpu.async_copy(x_hbm.at[src_v.at[b]], rows[b], gsem.at[b])

        def round_body(r, carry):
            for b in range(_K):
                j = r * _K + b
                b2 = (b + 2) % _K
                pltpu.make_async_copy(x_hbm.at[src_v.at[j]], rows[b],
                                      gsem.at[b]).wait()
                pltpu.async_copy(rows[b], acc_sh.at[dst_v.at[j]], ssem.at[b],
                                 add=True)

                @pl.when(jnp.logical_and(j >= 2, j + _K - 2 < cpt))
                def _refill():
                    pltpu.make_async_copy(rows[b2], acc_sh.at[dst_v.at[j - 2]],
                                          ssem.at[b2]).wait()
                    pltpu.async_copy(x_hbm.at[src_v.at[j + _K - 2]], rows[b2],
                                    gsem.at[b2])
            return carry

        lax.fori_loop(0, cpt // _K, round_body, 0)
        for b in range(_K):
            pltpu.make_async_copy(rows[b], acc_sh.at[dst_v.at[cpt - _K + b]],
                                  ssem.at[b]).wait()
        plsc.subcore_barrier()
        pltpu.sync_copy(acc_sh.at[pl.ds(s * rpt, rpt)],
                        out_hbm.at[c, pl.ds(s * rpt, rpt)])

    return agg_kernel


def _normb_from_deg(deg_ref, d):
    dsum = deg_ref[0:1, :] + deg_ref[1:2, :]
    norm = jnp.where(dsum > 0, lax.rsqrt(jnp.maximum(dsum, 1.0)), 0.0)
    ones = jnp.ones((1, d), jnp.float32)
    return lax.dot_general(norm, ones, (((0,), (0,)), ((), ())),
                           preferred_element_type=jnp.float32)


def _scale_kernel(deg_ref, x_ref, o_ref):
    normb = _normb_from_deg(deg_ref, x_ref.shape[1])
    o_ref[...] = (x_ref[...] * normb).astype(jnp.bfloat16)


def _layer1_kernel(deg_ref, p_ref, w_ref, b_ref, o_ref):
    normb = _normb_from_deg(deg_ref, p_ref.shape[2])
    agg = (p_ref[0].astype(jnp.float32) + p_ref[1].astype(jnp.float32)) * normb
    h = jnp.dot(agg, w_ref[...], preferred_element_type=jnp.float32) + b_ref[...]
    o_ref[...] = (jnp.maximum(h, 0.0) * normb).astype(jnp.bfloat16)


def _layer2_kernel(deg_ref, p_ref, w2_ref, b2_ref, w3_ref, b3_ref,
                   o_ref, acc_ref, *, nblocks, n_nodes, blk):
    i = pl.program_id(0)
    normb = _normb_from_deg(deg_ref, p_ref.shape[2])
    agg = (p_ref[0].astype(jnp.float32) + p_ref[1].astype(jnp.float32)) * normb
    h = jnp.dot(agg, w2_ref[...], preferred_element_type=jnp.float32) + b2_ref[...]
    h = jnp.maximum(h, 0.0)
    row = lax.broadcasted_iota(jnp.int32, h.shape, 0) + i * blk
    h = jnp.where(row < n_nodes, h, 0.0)
    csum = jnp.sum(h, axis=0, keepdims=True)

    @pl.when(i == 0)
    def _init():
        acc_ref[...] = jnp.zeros_like(acc_ref)

    acc_ref[...] += csum

    @pl.when(i == nblocks - 1)
    def _final():
        hg = acc_ref[...] * (1.0 / n_nodes)
        o_ref[...] = (jnp.dot(hg, w3_ref[...], preferred_element_type=jnp.float32)
                      + b3_ref[...])


def kernel(x, edge_index, W1, b1, W2, b2, W3, b3):
    n, din = x.shape
    e = edge_index.shape[1]
    hid = W1.shape[1]
    out2 = W2.shape[1]
    ncls = W3.shape[1]
    nch = e // _C

    BLK = 2048
    npad = -(-n // BLK) * BLK

    src2d = edge_index[0].reshape(nch, _C)
    dst2d = edge_index[1].reshape(nch, _C)
    zeros1 = jnp.zeros((npad // _NZ,), jnp.float32)
    zerosf = jnp.zeros((npad // _NS, din), jnp.bfloat16)
    xp = jnp.pad(x, ((0, npad - n), (0, 0)))

    deg = _make_deg(npad, nch)(dst2d, zeros1)

    gridn = npad // BLK

    xs = pl.pallas_call(
        _scale_kernel,
        grid=(gridn,),
        in_specs=[
            pl.BlockSpec((_NC, BLK), lambda i: (0, i)),
            pl.BlockSpec((BLK, din), lambda i: (i, 0)),
        ],
        out_specs=pl.BlockSpec((BLK, din), lambda i: (i, 0)),
        out_shape=jax.ShapeDtypeStruct((npad, din), jnp.bfloat16),
    )(deg, xp)

    p1 = _make_agg(npad, nch, din)(xs, src2d, dst2d, zerosf)

    h1s = pl.pallas_call(
        _layer1_kernel,
        grid=(gridn,),
        in_specs=[
            pl.BlockSpec((_NC, BLK), lambda i: (0, i)),
            pl.BlockSpec((_NC, BLK, din), lambda i: (0, i, 0)),
            pl.BlockSpec((din, hid), lambda i: (0, 0)),
            pl.BlockSpec((1, hid), lambda i: (0, 0)),
        ],
        out_specs=pl.BlockSpec((BLK, hid), lambda i: (i, 0)),
        out_shape=jax.ShapeDtypeStruct((npad, hid), jnp.bfloat16),
    )(deg, p1, W1, b1.reshape(1, hid))

    p2 = _make_agg(npad, nch, hid)(h1s, src2d, dst2d, zerosf)

    y = pl.pallas_call(
        functools.partial(_layer2_kernel, nblocks=gridn, n_nodes=n, blk=BLK),
        grid=(gridn,),
        in_specs=[
            pl.BlockSpec((_NC, BLK), lambda i: (0, i)),
            pl.BlockSpec((_NC, BLK, hid), lambda i: (0, i, 0)),
            pl.BlockSpec((hid, out2), lambda i: (0, 0)),
            pl.BlockSpec((1, out2), lambda i: (0, 0)),
            pl.BlockSpec((out2, ncls), lambda i: (0, 0)),
            pl.BlockSpec((1, ncls), lambda i: (0, 0)),
        ],
        out_specs=pl.BlockSpec((1, ncls), lambda i: (0, 0)),
        out_shape=jax.ShapeDtypeStruct((1, ncls), jnp.float32),
        scratch_shapes=[pltpu.VMEM((1, out2), jnp.float32)],
    )(deg, p2, W2, b2.reshape(1, out2), W3, b3.reshape(1, ncls))

    return y
